# Initial kernel scaffold; baseline (speedup 1.0000x reference)
import jax
import jax.numpy as jnp
from jax.experimental import pallas as pl


def kernel(hidden_states, W_gate, e_bias, W_gate_up, W_down, Ws_gate_up, Ws_down):
    raise NotImplementedError("write your pallas kernel here")


# speedup vs baseline: 2.0351x; 2.0351x over previous
import functools

import jax
import jax.numpy as jnp
from jax import lax
from jax.experimental import pallas as pl
from jax.experimental.pallas import tpu as pltpu

_T, _D, _E, _DFF, _NG, _TG, _K, _NS = 2048, 1024, 8, 512, 4, 2, 2, 2
_RSF = 2.5
_NEG = float(jnp.finfo(jnp.float32).min)


def _routing_body(x_ref, wg_ref, eb_ref, comb_ref):
    x = x_ref[...]
    logits = jnp.dot(x, wg_ref[...], preferred_element_type=jnp.float32)
    scores = jax.nn.sigmoid(logits)
    sc = scores + eb_ref[...]
    t = x.shape[0]
    ge = lax.broadcasted_iota(jnp.int32, (_E, _NG), 0)
    gg = lax.broadcasted_iota(jnp.int32, (_E, _NG), 1)
    gmat = (ge // (_E // _NG) == gg).astype(jnp.float32)
    sc_h = sc.astype(jnp.bfloat16).astype(jnp.float32)
    sc_m = (sc - sc_h).astype(jnp.bfloat16).astype(jnp.float32)
    sc_l = sc - sc_h - sc_m
    gs = (jnp.dot(sc_h, gmat, preferred_element_type=jnp.float32)
          + jnp.dot(sc_m, gmat, preferred_element_type=jnp.float32)
          + jnp.dot(sc_l, gmat, preferred_element_type=jnp.float32))
    ii4 = lax.broadcasted_iota(jnp.int32, (t, _NG), 1)
    m1 = jnp.max(gs, axis=1, keepdims=True)
    im1 = jnp.min(jnp.where(gs == m1, ii4, _NG), axis=1, keepdims=True)
    gs2 = jnp.where(ii4 == im1, _NEG, gs)
    m2 = jnp.max(gs2, axis=1, keepdims=True)
    im2 = jnp.min(jnp.where(gs2 == m2, ii4, _NG), axis=1, keepdims=True)
    eg = lax.broadcasted_iota(jnp.int32, (t, _E), 1) // (_E // _NG)
    emask = (eg == im1) | (eg == im2)
    masked = jnp.where(emask, sc, _NEG)
    ii8 = lax.broadcasted_iota(jnp.int32, (t, _E), 1)
    mm1 = jnp.max(masked, axis=1, keepdims=True)
    ie1 = jnp.min(jnp.where(masked == mm1, ii8, _E), axis=1, keepdims=True)
    masked2 = jnp.where(ii8 == ie1, _NEG, masked)
    mm2 = jnp.max(masked2, axis=1, keepdims=True)
    ie2 = jnp.min(jnp.where(masked2 == mm2, ii8, _E), axis=1, keepdims=True)
    w1 = jnp.sum(jnp.where(ii8 == ie1, scores, 0.0), axis=1, keepdims=True)
    w2 = jnp.sum(jnp.where(ii8 == ie2, scores, 0.0), axis=1, keepdims=True)
    den = w1 + w2 + 1e-20
    w1n = w1 / den * _RSF
    w2n = w2 / den * _RSF
    comb_ref[...] = (jnp.where(ii8 == ie1, w1n, 0.0)
                     + jnp.where(ii8 == ie2, w2n, 0.0))


def _dense_moe_body(x_ref, comb_ref, wgu_ref, wd_ref, out_ref, acc_ref):
    e = pl.program_id(0)
    t = pl.program_id(1)
    bt = out_ref.shape[0]
    x = x_ref[pl.ds(t * bt, bt), :]
    gu = jnp.dot(x, wgu_ref[0], preferred_element_type=jnp.float32)
    g = gu[:, :_DFF]
    u = gu[:, _DFF:]
    h = (g * jax.nn.sigmoid(g)) * u
    y = jnp.dot(h, wd_ref[0], preferred_element_type=jnp.float32)
    ii = lax.broadcasted_iota(jnp.int32, (bt, _E), 1)
    comb_blk = comb_ref[pl.ds(t * bt, bt), :]
    c = jnp.sum(jnp.where(ii == e, comb_blk, 0.0), axis=1, keepdims=True)
    contrib = c * y

    @pl.when(e == 0)
    def _():
        acc_ref[pl.ds(t * bt, bt), :] = contrib

    @pl.when(e > 0)
    def _():
        acc_ref[pl.ds(t * bt, bt), :] = acc_ref[pl.ds(t * bt, bt), :] + contrib

    out_ref[...] = acc_ref[pl.ds(t * bt, bt), :]


def _shared_body(x_ref, ys_ref, wsgu_ref, wsd_ref, out_ref):
    x = x_ref[...]
    gu = jnp.dot(x, wsgu_ref[...], preferred_element_type=jnp.float32)
    half = _DFF * _NS
    g = gu[:, :half]
    u = gu[:, half:]
    h = (g * jax.nn.sigmoid(g)) * u
    out_ref[...] = ys_ref[...] + jnp.dot(h, wsd_ref[...],
                                         preferred_element_type=jnp.float32)


def kernel(hidden_states, W_gate, e_bias, W_gate_up, W_down, Ws_gate_up, Ws_down):
    x = hidden_states
    eb = e_bias.reshape(1, _E)

    combine = pl.pallas_call(
        _routing_body,
        out_shape=jax.ShapeDtypeStruct((_T, _E), jnp.float32),
    )(x, W_gate, eb)

    bt = 256
    nt = _T // bt
    routed = pl.pallas_call(
        _dense_moe_body,
        grid=(_E, nt),
        in_specs=[
            pl.BlockSpec((_T, _D), lambda e, t: (0, 0)),
            pl.BlockSpec((_T, _E), lambda e, t: (0, 0)),
            pl.BlockSpec((1, _D, 2 * _DFF), lambda e, t: (e, 0, 0)),
            pl.BlockSpec((1, _DFF, _D), lambda e, t: (e, 0, 0)),
        ],
        out_specs=pl.BlockSpec((bt, _D), lambda e, t: (t, 0)),
        out_shape=jax.ShapeDtypeStruct((_T, _D), jnp.float32),
        scratch_shapes=[pltpu.VMEM((_T, _D), jnp.float32)],
    )(x, combine, W_gate_up, W_down)

    bt2 = 256
    out = pl.pallas_call(
        _shared_body,
        grid=(_T // bt2,),
        in_specs=[
            pl.BlockSpec((bt2, _D), lambda t: (t, 0)),
            pl.BlockSpec((bt2, _D), lambda t: (t, 0)),
            pl.BlockSpec((_D, 2 * _DFF * _NS), lambda t: (0, 0)),
            pl.BlockSpec((_DFF * _NS, _D), lambda t: (0, 0)),
        ],
        out_specs=pl.BlockSpec((bt2, _D), lambda t: (t, 0)),
        out_shape=jax.ShapeDtypeStruct((_T, _D), jnp.float32),
    )(x, routed, Ws_gate_up, Ws_down)
    return out
